# baseline (speedup 1.0000x reference)
import jax
import jax.numpy as jnp
from jax.experimental import pallas as pl
from jax.experimental.pallas import tpu as pltpu

_OUT_FEATURES = 10
_BATCH_TILE = 16384


def _linear_t_kernel(x_ref, w_ref, b_ref, o_ref):
    acc = jnp.dot(x_ref[...], w_ref[...], preferred_element_type=jnp.float32)
    acc = acc + b_ref[...]
    o_ref[...] = jnp.transpose(acc).astype(o_ref.dtype)


def kernel(x, w_padded, b_padded):
    B, in_f = x.shape
    out_f = _OUT_FEATURES
    w = w_padded[:, :out_f]
    b = b_padded[:, :out_f]

    tb = min(_BATCH_TILE, B)
    b_rows = pl.cdiv(B, tb) * tb
    x_p = x if b_rows == B else jnp.pad(x, ((0, b_rows - B), (0, 0)))

    yt = pl.pallas_call(
        _linear_t_kernel,
        out_shape=jax.ShapeDtypeStruct((out_f, b_rows), x.dtype),
        grid=(b_rows // tb,),
        in_specs=[
            pl.BlockSpec((tb, in_f), lambda i: (i, 0)),
            pl.BlockSpec((in_f, out_f), lambda i: (0, 0)),
            pl.BlockSpec((1, out_f), lambda i: (0, 0)),
        ],
        out_specs=pl.BlockSpec((out_f, tb), lambda i: (0, i)),
        compiler_params=pltpu.CompilerParams(
            dimension_semantics=("parallel",)),
    )(x_p, w, b)
    y = yt.T
    return y if b_rows == B else y[:B]


# speedup vs baseline: 1.0363x; 1.0363x over previous
import jax
import jax.numpy as jnp
from jax.experimental import pallas as pl
from jax.experimental.pallas import tpu as pltpu

_OUT_FEATURES = 10
_BATCH_TILE = 32768


def _linear_t_kernel(x_ref, w_ref, b_ref, o_ref):
    acc = jnp.dot(x_ref[...], w_ref[...], preferred_element_type=jnp.float32)
    acc = acc + b_ref[...]
    o_ref[...] = jnp.transpose(acc).astype(o_ref.dtype)


def kernel(x, w_padded, b_padded):
    B, in_f = x.shape
    out_f = _OUT_FEATURES
    w = w_padded[:, :out_f]
    b = b_padded[:, :out_f]

    tb = min(_BATCH_TILE, B)
    b_rows = pl.cdiv(B, tb) * tb
    x_p = x if b_rows == B else jnp.pad(x, ((0, b_rows - B), (0, 0)))

    yt = pl.pallas_call(
        _linear_t_kernel,
        out_shape=jax.ShapeDtypeStruct((out_f, b_rows), x.dtype),
        grid=(b_rows // tb,),
        in_specs=[
            pl.BlockSpec((tb, in_f), lambda i: (i, 0)),
            pl.BlockSpec((in_f, out_f), lambda i: (0, 0)),
            pl.BlockSpec((1, out_f), lambda i: (0, 0)),
        ],
        out_specs=pl.BlockSpec((out_f, tb), lambda i: (0, i)),
        compiler_params=pltpu.CompilerParams(
            dimension_semantics=("parallel",)),
    )(x_p, w, b)
    y = yt.T
    return y if b_rows == B else y[:B]
